# baseline (speedup 1.0000x reference)
import functools
import math

import jax
import jax.numpy as jnp
from jax import lax
from jax.experimental import pallas as pl
from jax.experimental.pallas import tpu as pltpu
from jax.experimental.pallas import tpu_sc as plsc

TOP_K = 2


def _gate_body(S, E, C, x_ref, wg_ref, idx_ref, w_ref, xb_ref):
    x = x_ref[...]
    wg = wg_ref[...]
    xb_ref[...] = x.astype(jnp.bfloat16)
    logits = jnp.dot(x, wg, preferred_element_type=jnp.float32)
    m = jnp.max(logits, axis=-1, keepdims=True)
    p = jnp.exp(logits - m)
    gates = p / jnp.sum(p, axis=-1, keepdims=True)

    eids = lax.broadcasted_iota(jnp.int32, (S, E), 1)
    mx1 = jnp.max(gates, axis=-1, keepdims=True)
    idx1 = jnp.min(jnp.where(gates == mx1, eids, E), axis=-1)
    mask1 = eids == idx1[:, None]
    g1 = mx1[:, 0]
    gates2 = jnp.where(mask1, 0.0, gates)
    mx2 = jnp.max(gates2, axis=-1, keepdims=True)
    idx2 = jnp.min(jnp.where(gates2 == mx2, eids, E), axis=-1)
    mask2 = eids == idx2[:, None]
    g2 = mx2[:, 0]

    m1f = mask1.astype(jnp.float32)
    m2f = mask2.astype(jnp.float32)
    m12 = jnp.concatenate([m1f, m2f], axis=1).astype(jnp.bfloat16)
    BS = 128
    G = S // BS
    trilb = (lax.broadcasted_iota(jnp.int32, (BS, BS), 1)
             <= lax.broadcasted_iota(jnp.int32, (BS, BS), 0)).astype(jnp.bfloat16)
    pieces = []
    off = jnp.zeros((1, 2 * E), jnp.float32)
    for g in range(G):
        blk = m12[g * BS:(g + 1) * BS, :]
        cb = jnp.dot(trilb, blk, preferred_element_type=jnp.float32) + off
        pieces.append(cb)
        off = cb[BS - 1:BS, :]
    cum12 = jnp.concatenate(pieces, axis=0)
    cum1 = cum12[:, :E]
    cum2 = cum12[:, E:]
    cnt1 = off[:, :E]
    loc1 = (jnp.sum(jnp.where(mask1, cum1, 0.0), axis=-1) - 1.0).astype(jnp.int32)
    loc2 = (jnp.sum(jnp.where(mask2, cum2 + cnt1, 0.0), axis=-1) - 1.0).astype(jnp.int32)

    denom = g1 + g2 + 1e-9
    keep1 = loc1 < C
    keep2 = loc2 < C
    w1 = jnp.where(keep1, g1 / denom, 0.0)
    w2 = jnp.where(keep2, g2 / denom, 0.0)
    srow1 = jnp.where(keep1, idx1 * C + loc1, E * C)
    srow2 = jnp.where(keep2, idx2 * C + loc2, E * C)
    grow1 = idx1 * C + jnp.minimum(loc1, C - 1)
    grow2 = idx2 * C + jnp.minimum(loc2, C - 1)

    idx_ref[0, :] = srow1
    idx_ref[1, :] = srow2
    idx_ref[2, :] = grow1
    idx_ref[3, :] = grow2
    w_ref[0, :, :] = jnp.broadcast_to(w1[:, None], (S, 16))
    w_ref[1, :, :] = jnp.broadcast_to(w2[:, None], (S, 16))


def _ffn_body(d_ref, w1_ref, b1_ref, w2_ref, b2_ref, o_ref):
    d = d_ref[0]
    h = jnp.maximum(
        jnp.dot(d, w1_ref[0], preferred_element_type=jnp.float32) + b1_ref[0],
        0.0)
    o_ref[0] = jnp.dot(h, w2_ref[0], preferred_element_type=jnp.float32) + b2_ref[0]


def _make_dispatch(S, M, n_rows):
    info = plsc.get_sparse_core_info()
    nw = info.num_cores * info.num_subcores
    tok_w = S // nw
    ht = tok_w // 2
    mesh = plsc.VectorSubcoreMesh(core_axis_name="c", subcore_axis_name="s")

    @functools.partial(
        pl.kernel,
        out_type=jax.ShapeDtypeStruct((n_rows, M), jnp.int32),
        mesh=mesh,
        scratch_types=[
            pltpu.VMEM((2, ht, M), jnp.int32),
            pltpu.VMEM((ht,), jnp.int32),
            pltpu.VMEM((ht,), jnp.int32),
            pltpu.VMEM((ht,), jnp.int32),
            pltpu.VMEM((ht,), jnp.int32),
            pltpu.SemaphoreType.DMA,
            pltpu.SemaphoreType.DMA,
            pltpu.SemaphoreType.DMA,
        ],
    )
    def dispatch(x_hbm, idx_hbm, out_hbm, src_v, i1a_v, i1b_v, i2a_v, i2b_v,
                 sem0, sem1, ssem):
        wid = lax.axis_index("s") * info.num_cores + lax.axis_index("c")
        base = wid * tok_w
        sems = (sem0, sem1)
        ld0 = pltpu.async_copy(x_hbm.at[pl.ds(base, ht)], src_v.at[0], sems[0])
        ld1 = pltpu.async_copy(x_hbm.at[pl.ds(base + ht, ht)], src_v.at[1], sems[1])
        pltpu.sync_copy(idx_hbm.at[0, pl.ds(base, ht)], i1a_v)
        pltpu.sync_copy(idx_hbm.at[0, pl.ds(base + ht, ht)], i1b_v)
        pltpu.sync_copy(idx_hbm.at[1, pl.ds(base, ht)], i2a_v)
        pltpu.sync_copy(idx_hbm.at[1, pl.ds(base + ht, ht)], i2b_v)
        scats = []
        for r, ld, ia, ib in ((0, ld0, i1a_v, i2a_v), (1, ld1, i1b_v, i2b_v)):
            ld.wait()
            scats.append(pltpu.async_copy(src_v.at[r], out_hbm.at[ia], ssem))
            scats.append(pltpu.async_copy(src_v.at[r], out_hbm.at[ib], ssem))
        for cp in scats:
            cp.wait()

    return dispatch


def _make_combine(S, M, n_rows):
    info = plsc.get_sparse_core_info()
    nw = info.num_cores * info.num_subcores
    tok_w = S // nw
    ht = 16
    rounds = tok_w // ht
    nv = M // 16
    mesh = plsc.VectorSubcoreMesh(core_axis_name="c", subcore_axis_name="s")

    @functools.partial(
        pl.kernel,
        out_type=jax.ShapeDtypeStruct((S, M), jnp.float32),
        mesh=mesh,
        scratch_types=[
            pltpu.VMEM((2, ht, M), jnp.float32),
            pltpu.VMEM((2, ht, M), jnp.float32),
            pltpu.VMEM((tok_w,), jnp.int32),
            pltpu.VMEM((tok_w,), jnp.int32),
            pltpu.VMEM((tok_w, 16), jnp.float32),
            pltpu.VMEM((tok_w, 16), jnp.float32),
            pltpu.SemaphoreType.DMA,
            pltpu.SemaphoreType.DMA,
            pltpu.SemaphoreType.DMA,
            pltpu.SemaphoreType.DMA,
        ],
    )
    def combine(eo_hbm, idx_hbm, w_hbm, out_hbm,
                buf1_v, buf2_v, i1_v, i2_v, w1_v, w2_v,
                sem0, sem1, osem, psem):
        wid = lax.axis_index("s") * info.num_cores + lax.axis_index("c")
        base = wid * tok_w
        sems = (sem0, sem1)

        pltpu.sync_copy(idx_hbm.at[2, pl.ds(base, tok_w)], i1_v)
        pltpu.sync_copy(idx_hbm.at[3, pl.ds(base, tok_w)], i2_v)
        lw1 = pltpu.async_copy(w_hbm.at[0, pl.ds(base, tok_w), :], w1_v, psem)
        lw2 = pltpu.async_copy(w_hbm.at[1, pl.ds(base, tok_w), :], w2_v, psem)

        def gather(r, b):
            sl = pl.ds(r * ht, ht)
            return (pltpu.async_copy(eo_hbm.at[i1_v.at[sl]], buf1_v.at[b], sems[b]),
                    pltpu.async_copy(eo_hbm.at[i2_v.at[sl]], buf2_v.at[b], sems[b]))

        pend = gather(0, 0)
        lw1.wait()
        lw2.wait()
        owaits = []
        for r in range(rounds):
            b = r % 2
            pend[0].wait()
            pend[1].wait()
            if r + 1 < rounds:
                if owaits:
                    owaits.pop(0).wait()
                pend = gather(r + 1, 1 - b)

            def body(t, _):
                wv1 = w1_v[r * ht + t, :]
                wv2 = w2_v[r * ht + t, :]
                for j in range(nv):
                    sl = pl.ds(j * 16, 16)
                    buf1_v[b, t, sl] = wv1 * buf1_v[b, t, sl] + wv2 * buf2_v[b, t, sl]
                return 0

            lax.fori_loop(0, ht, body, 0)
            owaits.append(pltpu.async_copy(
                buf1_v.at[b], out_hbm.at[pl.ds(base + r * ht, ht)], osem))
        for cp in owaits:
            cp.wait()

    return combine


def kernel(x, Wg, W1, b1, W2, b2):
    B, T, M = x.shape
    S = B * T
    E = Wg.shape[1]
    F = W1.shape[2]
    C = int(math.ceil(TOP_K * S / E))
    xf = x.reshape(S, M)

    idx_out, w_out, xb = pl.pallas_call(
        functools.partial(_gate_body, S, E, C),
        out_shape=[
            jax.ShapeDtypeStruct((4, S), jnp.int32),
            jax.ShapeDtypeStruct((2, S, 16), jnp.float32),
            jax.ShapeDtypeStruct((S, M), jnp.bfloat16),
        ],
    )(xf, Wg)

    n_rows = (E + 1) * C
    xb32 = lax.bitcast_convert_type(xb.reshape(S, M // 2, 2), jnp.int32)
    disp = _make_dispatch(S, M // 2, n_rows)(xb32, idx_out)
    disp3 = lax.bitcast_convert_type(
        disp.reshape(E + 1, C, M // 2), jnp.bfloat16).reshape(E + 1, C, M)

    eo = pl.pallas_call(
        _ffn_body,
        grid=(E,),
        in_specs=[
            pl.BlockSpec((1, C, M), lambda e: (e, 0, 0)),
            pl.BlockSpec((1, M, F), lambda e: (e, 0, 0)),
            pl.BlockSpec((1, 1, F), lambda e: (e, 0, 0)),
            pl.BlockSpec((1, F, M), lambda e: (e, 0, 0)),
            pl.BlockSpec((1, 1, M), lambda e: (e, 0, 0)),
        ],
        out_specs=pl.BlockSpec((1, C, M), lambda e: (e, 0, 0)),
        out_shape=jax.ShapeDtypeStruct((E, C, M), jnp.float32),
        compiler_params=pltpu.CompilerParams(vmem_limit_bytes=100 * 1024 * 1024),
    )(disp3, W1, b1.reshape(E, 1, F), W2, b2.reshape(E, 1, M))

    out = _make_combine(S, M, E * C)(eo.reshape(E * C, M), idx_out, w_out)
    return out.reshape(B, T, M)


# speedup vs baseline: 1.7624x; 1.7624x over previous
import functools
import math

import jax
import jax.numpy as jnp
from jax import lax
from jax.experimental import pallas as pl
from jax.experimental.pallas import tpu as pltpu
from jax.experimental.pallas import tpu_sc as plsc

TOP_K = 2


def _gate_body(S, E, C, x_ref, wg_ref, idx_ref, w_ref):
    x = x_ref[...]
    wg = wg_ref[...]
    logits = jnp.dot(x, wg, preferred_element_type=jnp.float32)
    m = jnp.max(logits, axis=-1, keepdims=True)
    p = jnp.exp(logits - m)
    gates = p / jnp.sum(p, axis=-1, keepdims=True)

    eids = lax.broadcasted_iota(jnp.int32, (S, E), 1)
    mx1 = jnp.max(gates, axis=-1, keepdims=True)
    idx1 = jnp.min(jnp.where(gates == mx1, eids, E), axis=-1)
    mask1 = eids == idx1[:, None]
    g1 = mx1[:, 0]
    gates2 = jnp.where(mask1, 0.0, gates)
    mx2 = jnp.max(gates2, axis=-1, keepdims=True)
    idx2 = jnp.min(jnp.where(gates2 == mx2, eids, E), axis=-1)
    mask2 = eids == idx2[:, None]
    g2 = mx2[:, 0]

    m1f = mask1.astype(jnp.float32)
    m2f = mask2.astype(jnp.float32)
    m12 = jnp.concatenate([m1f, m2f], axis=1).astype(jnp.bfloat16)
    BS = 128
    G = S // BS
    trilb = (lax.broadcasted_iota(jnp.int32, (BS, BS), 1)
             <= lax.broadcasted_iota(jnp.int32, (BS, BS), 0)).astype(jnp.bfloat16)
    pieces = []
    off = jnp.zeros((1, 2 * E), jnp.float32)
    for g in range(G):
        blk = m12[g * BS:(g + 1) * BS, :]
        cb = jnp.dot(trilb, blk, preferred_element_type=jnp.float32) + off
        pieces.append(cb)
        off = cb[BS - 1:BS, :]
    cum12 = jnp.concatenate(pieces, axis=0)
    cum1 = cum12[:, :E]
    cum2 = cum12[:, E:]
    cnt1 = off[:, :E]
    loc1 = (jnp.sum(jnp.where(mask1, cum1, 0.0), axis=-1) - 1.0).astype(jnp.int32)
    loc2 = (jnp.sum(jnp.where(mask2, cum2 + cnt1, 0.0), axis=-1) - 1.0).astype(jnp.int32)

    denom = g1 + g2 + 1e-9
    keep1 = loc1 < C
    keep2 = loc2 < C
    w1 = jnp.where(keep1, g1 / denom, 0.0)
    w2 = jnp.where(keep2, g2 / denom, 0.0)
    srow1 = jnp.where(keep1, idx1 * C + loc1, E * C)
    srow2 = jnp.where(keep2, idx2 * C + loc2, E * C)
    grow1 = idx1 * C + jnp.minimum(loc1, C - 1)
    grow2 = idx2 * C + jnp.minimum(loc2, C - 1)

    idx_ref[0, :] = srow1
    idx_ref[1, :] = srow2
    idx_ref[2, :] = grow1
    idx_ref[3, :] = grow2
    w_ref[0, :, :] = jnp.broadcast_to(w1[:, None], (S, 16))
    w_ref[1, :, :] = jnp.broadcast_to(w2[:, None], (S, 16))


def _ffn_body(d_ref, w1_ref, b1_ref, w2_ref, b2_ref, o_ref):
    d = d_ref[0]
    h = jnp.maximum(
        jnp.dot(d, w1_ref[0], preferred_element_type=jnp.float32) + b1_ref[0],
        0.0)
    o_ref[0] = jnp.dot(h, w2_ref[0], preferred_element_type=jnp.float32) + b2_ref[0]


def _make_dispatch(S, M, n_rows):
    info = plsc.get_sparse_core_info()
    nw = info.num_cores * info.num_subcores
    tok_w = S // nw
    ht = tok_w // 2
    mesh = plsc.VectorSubcoreMesh(core_axis_name="c", subcore_axis_name="s")

    @functools.partial(
        pl.kernel,
        out_type=jax.ShapeDtypeStruct((n_rows, M), jnp.float32),
        mesh=mesh,
        scratch_types=[
            pltpu.VMEM((2, ht, M), jnp.float32),
            pltpu.VMEM((ht,), jnp.int32),
            pltpu.VMEM((ht,), jnp.int32),
            pltpu.VMEM((ht,), jnp.int32),
            pltpu.VMEM((ht,), jnp.int32),
            pltpu.SemaphoreType.DMA,
            pltpu.SemaphoreType.DMA,
            pltpu.SemaphoreType.DMA,
        ],
    )
    def dispatch(x_hbm, idx_hbm, out_hbm, src_v, i1a_v, i1b_v, i2a_v, i2b_v,
                 sem0, sem1, ssem):
        wid = lax.axis_index("s") * info.num_cores + lax.axis_index("c")
        base = wid * tok_w
        sems = (sem0, sem1)
        ld0 = pltpu.async_copy(x_hbm.at[pl.ds(base, ht)], src_v.at[0], sems[0])
        ld1 = pltpu.async_copy(x_hbm.at[pl.ds(base + ht, ht)], src_v.at[1], sems[1])
        pltpu.sync_copy(idx_hbm.at[0, pl.ds(base, ht)], i1a_v)
        pltpu.sync_copy(idx_hbm.at[0, pl.ds(base + ht, ht)], i1b_v)
        pltpu.sync_copy(idx_hbm.at[1, pl.ds(base, ht)], i2a_v)
        pltpu.sync_copy(idx_hbm.at[1, pl.ds(base + ht, ht)], i2b_v)
        scats = []
        for r, ld, ia, ib in ((0, ld0, i1a_v, i2a_v), (1, ld1, i1b_v, i2b_v)):
            ld.wait()
            scats.append(pltpu.async_copy(src_v.at[r], out_hbm.at[ia], ssem))
            scats.append(pltpu.async_copy(src_v.at[r], out_hbm.at[ib], ssem))
        for cp in scats:
            cp.wait()

    return dispatch


def _make_combine(S, M, n_rows):
    info = plsc.get_sparse_core_info()
    nw = info.num_cores * info.num_subcores
    tok_w = S // nw
    ht = 16
    rounds = tok_w // ht
    nv = M // 16
    mesh = plsc.VectorSubcoreMesh(core_axis_name="c", subcore_axis_name="s")

    @functools.partial(
        pl.kernel,
        out_type=jax.ShapeDtypeStruct((S, M), jnp.float32),
        mesh=mesh,
        scratch_types=[
            pltpu.VMEM((2, ht, M), jnp.float32),
            pltpu.VMEM((2, ht, M), jnp.float32),
            pltpu.VMEM((tok_w,), jnp.int32),
            pltpu.VMEM((tok_w,), jnp.int32),
            pltpu.VMEM((tok_w, 16), jnp.float32),
            pltpu.VMEM((tok_w, 16), jnp.float32),
            pltpu.SemaphoreType.DMA,
            pltpu.SemaphoreType.DMA,
            pltpu.SemaphoreType.DMA,
            pltpu.SemaphoreType.DMA,
        ],
    )
    def combine(eo_hbm, idx_hbm, w_hbm, out_hbm,
                buf1_v, buf2_v, i1_v, i2_v, w1_v, w2_v,
                sem0, sem1, osem, psem):
        wid = lax.axis_index("s") * info.num_cores + lax.axis_index("c")
        base = wid * tok_w
        sems = (sem0, sem1)

        pltpu.sync_copy(idx_hbm.at[2, pl.ds(base, tok_w)], i1_v)
        pltpu.sync_copy(idx_hbm.at[3, pl.ds(base, tok_w)], i2_v)
        lw1 = pltpu.async_copy(w_hbm.at[0, pl.ds(base, tok_w), :], w1_v, psem)
        lw2 = pltpu.async_copy(w_hbm.at[1, pl.ds(base, tok_w), :], w2_v, psem)

        def gather(r, b):
            sl = pl.ds(r * ht, ht)
            return (pltpu.async_copy(eo_hbm.at[i1_v.at[sl]], buf1_v.at[b], sems[b]),
                    pltpu.async_copy(eo_hbm.at[i2_v.at[sl]], buf2_v.at[b], sems[b]))

        pend = gather(0, 0)
        lw1.wait()
        lw2.wait()
        owaits = []
        for r in range(rounds):
            b = r % 2
            pend[0].wait()
            pend[1].wait()
            if r + 1 < rounds:
                if owaits:
                    owaits.pop(0).wait()
                pend = gather(r + 1, 1 - b)

            def body(t, _):
                wv1 = w1_v[r * ht + t, :]
                wv2 = w2_v[r * ht + t, :]
                for j in range(nv):
                    sl = pl.ds(j * 16, 16)
                    buf1_v[b, t, sl] = wv1 * buf1_v[b, t, sl] + wv2 * buf2_v[b, t, sl]
                return 0

            lax.fori_loop(0, ht, body, 0)
            owaits.append(pltpu.async_copy(
                buf1_v.at[b], out_hbm.at[pl.ds(base + r * ht, ht)], osem))
        for cp in owaits:
            cp.wait()

    return combine


def kernel(x, Wg, W1, b1, W2, b2):
    B, T, M = x.shape
    S = B * T
    E = Wg.shape[1]
    F = W1.shape[2]
    C = int(math.ceil(TOP_K * S / E))
    xf = x.reshape(S, M)

    idx_out, w_out = pl.pallas_call(
        functools.partial(_gate_body, S, E, C),
        out_shape=[
            jax.ShapeDtypeStruct((4, S), jnp.int32),
            jax.ShapeDtypeStruct((2, S, 16), jnp.float32),
        ],
    )(xf, Wg)

    n_rows = (E + 1) * C
    disp = _make_dispatch(S, M, n_rows)(xf, idx_out)
    disp3 = disp.reshape(E + 1, C, M)

    eo = pl.pallas_call(
        _ffn_body,
        grid=(E,),
        in_specs=[
            pl.BlockSpec((1, C, M), lambda e: (e, 0, 0)),
            pl.BlockSpec((1, M, F), lambda e: (e, 0, 0)),
            pl.BlockSpec((1, 1, F), lambda e: (e, 0, 0)),
            pl.BlockSpec((1, F, M), lambda e: (e, 0, 0)),
            pl.BlockSpec((1, 1, M), lambda e: (e, 0, 0)),
        ],
        out_specs=pl.BlockSpec((1, C, M), lambda e: (e, 0, 0)),
        out_shape=jax.ShapeDtypeStruct((E, C, M), jnp.float32),
        compiler_params=pltpu.CompilerParams(vmem_limit_bytes=100 * 1024 * 1024),
    )(disp3, W1, b1.reshape(E, 1, F), W2, b2.reshape(E, 1, M))

    out = _make_combine(S, M, E * C)(eo.reshape(E * C, M), idx_out, w_out)
    return out.reshape(B, T, M)


# speedup vs baseline: 1.7625x; 1.0001x over previous
import functools
import math

import jax
import jax.numpy as jnp
from jax import lax
from jax.experimental import pallas as pl
from jax.experimental.pallas import tpu as pltpu
from jax.experimental.pallas import tpu_sc as plsc

TOP_K = 2


def _gate_body(S, E, C, x_ref, wg_ref, idx_ref, w_ref):
    x = x_ref[...]
    wg = wg_ref[...]
    logits = jnp.dot(x, wg, preferred_element_type=jnp.float32)
    m = jnp.max(logits, axis=-1, keepdims=True)
    p = jnp.exp(logits - m)
    gates = p / jnp.sum(p, axis=-1, keepdims=True)

    eids = lax.broadcasted_iota(jnp.int32, (S, E), 1)
    mx1 = jnp.max(gates, axis=-1, keepdims=True)
    idx1 = jnp.min(jnp.where(gates == mx1, eids, E), axis=-1)
    mask1 = eids == idx1[:, None]
    g1 = mx1[:, 0]
    gates2 = jnp.where(mask1, 0.0, gates)
    mx2 = jnp.max(gates2, axis=-1, keepdims=True)
    idx2 = jnp.min(jnp.where(gates2 == mx2, eids, E), axis=-1)
    mask2 = eids == idx2[:, None]
    g2 = mx2[:, 0]

    m1f = mask1.astype(jnp.float32)
    m2f = mask2.astype(jnp.float32)
    m12 = jnp.concatenate([m1f, m2f], axis=1).astype(jnp.bfloat16)
    BS = 128
    G = S // BS
    trilb = (lax.broadcasted_iota(jnp.int32, (BS, BS), 1)
             <= lax.broadcasted_iota(jnp.int32, (BS, BS), 0)).astype(jnp.bfloat16)
    pieces = []
    off = jnp.zeros((1, 2 * E), jnp.float32)
    for g in range(G):
        blk = m12[g * BS:(g + 1) * BS, :]
        cb = jnp.dot(trilb, blk, preferred_element_type=jnp.float32) + off
        pieces.append(cb)
        off = cb[BS - 1:BS, :]
    cum12 = jnp.concatenate(pieces, axis=0)
    cum1 = cum12[:, :E]
    cum2 = cum12[:, E:]
    cnt1 = off[:, :E]
    loc1 = (jnp.sum(jnp.where(mask1, cum1, 0.0), axis=-1) - 1.0).astype(jnp.int32)
    loc2 = (jnp.sum(jnp.where(mask2, cum2 + cnt1, 0.0), axis=-1) - 1.0).astype(jnp.int32)

    denom = g1 + g2 + 1e-9
    keep1 = loc1 < C
    keep2 = loc2 < C
    w1 = jnp.where(keep1, g1 / denom, 0.0)
    w2 = jnp.where(keep2, g2 / denom, 0.0)
    srow1 = jnp.where(keep1, idx1 * C + loc1, E * C)
    srow2 = jnp.where(keep2, idx2 * C + loc2, E * C)
    grow1 = idx1 * C + jnp.minimum(loc1, C - 1)
    grow2 = idx2 * C + jnp.minimum(loc2, C - 1)

    idx_ref[0, :] = srow1
    idx_ref[1, :] = srow2
    idx_ref[2, :] = grow1
    idx_ref[3, :] = grow2
    w_ref[0, :, :] = jnp.broadcast_to(w1[:, None], (S, 16))
    w_ref[1, :, :] = jnp.broadcast_to(w2[:, None], (S, 16))


def _ffn_body(d_ref, w1_ref, b1_ref, w2_ref, b2_ref, o_ref):
    d = d_ref[0]
    h = jnp.maximum(
        jnp.dot(d, w1_ref[0], preferred_element_type=jnp.float32) + b1_ref[0],
        0.0)
    o_ref[0] = jnp.dot(h, w2_ref[0], preferred_element_type=jnp.float32) + b2_ref[0]


def _make_dispatch(S, M, n_rows):
    info = plsc.get_sparse_core_info()
    nw = info.num_cores * info.num_subcores
    tok_w = S // nw
    ht = tok_w // 2
    mesh = plsc.VectorSubcoreMesh(core_axis_name="c", subcore_axis_name="s")

    @functools.partial(
        pl.kernel,
        out_type=jax.ShapeDtypeStruct((n_rows, M), jnp.float32),
        mesh=mesh,
        scratch_types=[
            pltpu.VMEM((2, ht, M), jnp.float32),
            pltpu.VMEM((ht,), jnp.int32),
            pltpu.VMEM((ht,), jnp.int32),
            pltpu.VMEM((ht,), jnp.int32),
            pltpu.VMEM((ht,), jnp.int32),
            pltpu.SemaphoreType.DMA,
            pltpu.SemaphoreType.DMA,
            pltpu.SemaphoreType.DMA,
            pltpu.SemaphoreType.DMA,
        ],
    )
    def dispatch(x_hbm, idx_hbm, out_hbm, src_v, i1a_v, i1b_v, i2a_v, i2b_v,
                 sem0, sem1, ssem, psem):
        wid = lax.axis_index("s") * info.num_cores + lax.axis_index("c")
        base = wid * tok_w
        sems = (sem0, sem1)
        ld0 = pltpu.async_copy(x_hbm.at[pl.ds(base, ht)], src_v.at[0], sems[0])
        ld1 = pltpu.async_copy(x_hbm.at[pl.ds(base + ht, ht)], src_v.at[1], sems[1])
        idx_lds = [
            pltpu.async_copy(idx_hbm.at[0, pl.ds(base, ht)], i1a_v, psem),
            pltpu.async_copy(idx_hbm.at[0, pl.ds(base + ht, ht)], i1b_v, psem),
            pltpu.async_copy(idx_hbm.at[1, pl.ds(base, ht)], i2a_v, psem),
            pltpu.async_copy(idx_hbm.at[1, pl.ds(base + ht, ht)], i2b_v, psem),
        ]
        for cp in idx_lds:
            cp.wait()
        scats = []
        for r, ld, ia, ib in ((0, ld0, i1a_v, i2a_v), (1, ld1, i1b_v, i2b_v)):
            ld.wait()
            scats.append(pltpu.async_copy(src_v.at[r], out_hbm.at[ia], ssem))
            scats.append(pltpu.async_copy(src_v.at[r], out_hbm.at[ib], ssem))
        for cp in scats:
            cp.wait()

    return dispatch


def _make_combine(S, M, n_rows):
    info = plsc.get_sparse_core_info()
    nw = info.num_cores * info.num_subcores
    tok_w = S // nw
    ht = 16
    rounds = tok_w // ht
    nv = M // 16
    mesh = plsc.VectorSubcoreMesh(core_axis_name="c", subcore_axis_name="s")

    @functools.partial(
        pl.kernel,
        out_type=jax.ShapeDtypeStruct((S, M), jnp.float32),
        mesh=mesh,
        scratch_types=[
            pltpu.VMEM((2, ht, M), jnp.float32),
            pltpu.VMEM((2, ht, M), jnp.float32),
            pltpu.VMEM((tok_w,), jnp.int32),
            pltpu.VMEM((tok_w,), jnp.int32),
            pltpu.VMEM((tok_w, 16), jnp.float32),
            pltpu.VMEM((tok_w, 16), jnp.float32),
            pltpu.SemaphoreType.DMA,
            pltpu.SemaphoreType.DMA,
            pltpu.SemaphoreType.DMA,
            pltpu.SemaphoreType.DMA,
        ],
    )
    def combine(eo_hbm, idx_hbm, w_hbm, out_hbm,
                buf1_v, buf2_v, i1_v, i2_v, w1_v, w2_v,
                sem0, sem1, osem, psem):
        wid = lax.axis_index("s") * info.num_cores + lax.axis_index("c")
        base = wid * tok_w
        sems = (sem0, sem1)

        pltpu.sync_copy(idx_hbm.at[2, pl.ds(base, tok_w)], i1_v)
        pltpu.sync_copy(idx_hbm.at[3, pl.ds(base, tok_w)], i2_v)
        lw1 = pltpu.async_copy(w_hbm.at[0, pl.ds(base, tok_w), :], w1_v, psem)
        lw2 = pltpu.async_copy(w_hbm.at[1, pl.ds(base, tok_w), :], w2_v, psem)

        def gather(r, b):
            sl = pl.ds(r * ht, ht)
            return (pltpu.async_copy(eo_hbm.at[i1_v.at[sl]], buf1_v.at[b], sems[b]),
                    pltpu.async_copy(eo_hbm.at[i2_v.at[sl]], buf2_v.at[b], sems[b]))

        pend = gather(0, 0)
        lw1.wait()
        lw2.wait()
        owaits = []
        for r in range(rounds):
            b = r % 2
            pend[0].wait()
            pend[1].wait()
            if r + 1 < rounds:
                if owaits:
                    owaits.pop(0).wait()
                pend = gather(r + 1, 1 - b)

            def body(t, _):
                wv1 = w1_v[r * ht + t, :]
                wv2 = w2_v[r * ht + t, :]
                for j in range(nv):
                    sl = pl.ds(j * 16, 16)
                    buf1_v[b, t, sl] = wv1 * buf1_v[b, t, sl] + wv2 * buf2_v[b, t, sl]
                return 0

            lax.fori_loop(0, ht, body, 0)
            owaits.append(pltpu.async_copy(
                buf1_v.at[b], out_hbm.at[pl.ds(base + r * ht, ht)], osem))
        for cp in owaits:
            cp.wait()

    return combine


def kernel(x, Wg, W1, b1, W2, b2):
    B, T, M = x.shape
    S = B * T
    E = Wg.shape[1]
    F = W1.shape[2]
    C = int(math.ceil(TOP_K * S / E))
    xf = x.reshape(S, M)

    idx_out, w_out = pl.pallas_call(
        functools.partial(_gate_body, S, E, C),
        out_shape=[
            jax.ShapeDtypeStruct((4, S), jnp.int32),
            jax.ShapeDtypeStruct((2, S, 16), jnp.float32),
        ],
    )(xf, Wg)

    n_rows = (E + 1) * C
    disp = _make_dispatch(S, M, n_rows)(xf, idx_out)
    disp3 = disp.reshape(E + 1, C, M)

    eo = pl.pallas_call(
        _ffn_body,
        grid=(E,),
        in_specs=[
            pl.BlockSpec((1, C, M), lambda e: (e, 0, 0)),
            pl.BlockSpec((1, M, F), lambda e: (e, 0, 0)),
            pl.BlockSpec((1, 1, F), lambda e: (e, 0, 0)),
            pl.BlockSpec((1, F, M), lambda e: (e, 0, 0)),
            pl.BlockSpec((1, 1, M), lambda e: (e, 0, 0)),
        ],
        out_specs=pl.BlockSpec((1, C, M), lambda e: (e, 0, 0)),
        out_shape=jax.ShapeDtypeStruct((E, C, M), jnp.float32),
        compiler_params=pltpu.CompilerParams(vmem_limit_bytes=100 * 1024 * 1024),
    )(disp3, W1, b1.reshape(E, 1, F), W2, b2.reshape(E, 1, M))

    out = _make_combine(S, M, E * C)(eo.reshape(E * C, M), idx_out, w_out)
    return out.reshape(B, T, M)
